# Initial kernel scaffold; baseline (speedup 1.0000x reference)
import jax
import jax.numpy as jnp
from jax.experimental import pallas as pl


def kernel(x, bn1_gamma, bn1_beta, bn1_mean, bn1_var, bn2_gamma, bn2_beta, bn2_mean, bn2_var, w1, w2, w_sc):
    raise NotImplementedError("write your pallas kernel here")


# speedup vs baseline: 31.6855x; 31.6855x over previous
import functools

import jax
import jax.numpy as jnp
from jax.experimental import pallas as pl
from jax.experimental.pallas import tpu as pltpu

_EPS = 1e-5
_VMEM_LIMIT = 48 * 1024 * 1024


def _block_body(xp_ref, s1_ref, b1_ref, w1_ref, s2_ref, b2_ref, w2_ref,
                wsc_ref, o_ref, *, ho, wo, cin, co):
    m = ho * wo
    f32 = jnp.float32

    a = xp_ref[...].astype(f32) * s1_ref[0] + b1_ref[0]
    a = jnp.maximum(a, 0.0).astype(jnp.bfloat16)

    pee = a[0, 0]
    poe = jnp.pad(a[1, 0], ((1, 0), (0, 0), (0, 0)))
    peo = jnp.pad(a[0, 1], ((0, 0), (1, 0), (0, 0)))
    poo = jnp.pad(a[1, 1], ((1, 0), (1, 0), (0, 0)))

    wins = (
        poo[0:ho, 0:wo], poe[0:ho, :], poo[0:ho, 1:wo + 1],
        peo[:, 0:wo], pee, peo[:, 1:wo + 1],
        poo[1:ho + 1, 0:wo], poe[1:ho + 1, :], poo[1:ho + 1, 1:wo + 1],
    )
    acc = jnp.zeros((m, co), f32)
    for t in range(9):
        acc = acc + jnp.dot(wins[t].reshape(m, cin), w1_ref[t],
                            preferred_element_type=f32)

    a2 = jnp.maximum(acc * s2_ref[0] + b2_ref[0], 0.0).astype(jnp.bfloat16)
    a2p = jnp.pad(a2.reshape(ho, wo, co), ((1, 1), (1, 1), (0, 0)))

    out = jnp.dot(pee.reshape(m, cin), wsc_ref[...], preferred_element_type=f32)

    for t in range(9):
        dy, dx = divmod(t, 3)
        win = a2p[dy:dy + ho, dx:dx + wo].reshape(m, co)
        out = out + jnp.dot(win, w2_ref[t], preferred_element_type=f32)

    o_ref[...] = out


def kernel(x, bn1_gamma, bn1_beta, bn1_mean, bn1_var,
           bn2_gamma, bn2_beta, bn2_mean, bn2_var, w1, w2, w_sc):
    n, cin, h, w = x.shape
    co = w1.shape[0]
    ho, wo = h // 2, w // 2
    m = ho * wo

    s1 = bn1_gamma / jnp.sqrt(bn1_var + _EPS)
    b1 = bn1_beta - bn1_mean * s1
    s2 = bn2_gamma / jnp.sqrt(bn2_var + _EPS)
    b2 = bn2_beta - bn2_mean * s2

    xp = x.transpose(0, 2, 3, 1).reshape(n, ho, 2, wo, 2, cin)
    xp = xp.transpose(0, 2, 4, 1, 3, 5).astype(jnp.bfloat16)

    wp1 = jnp.transpose(w1, (2, 3, 1, 0)).reshape(9, cin, co).astype(jnp.bfloat16)
    wp2 = jnp.transpose(w2, (2, 3, 1, 0)).reshape(9, co, co).astype(jnp.bfloat16)
    wsc = jnp.transpose(w_sc.reshape(co, cin), (1, 0)).astype(jnp.bfloat16)

    body = functools.partial(_block_body, ho=ho, wo=wo, cin=cin, co=co)
    out = pl.pallas_call(
        body,
        grid=(n,),
        in_specs=[
            pl.BlockSpec((None, 2, 2, ho, wo, cin),
                         lambda i: (i, 0, 0, 0, 0, 0)),
            pl.BlockSpec((1, cin), lambda i: (0, 0)),
            pl.BlockSpec((1, cin), lambda i: (0, 0)),
            pl.BlockSpec((9, cin, co), lambda i: (0, 0, 0)),
            pl.BlockSpec((1, co), lambda i: (0, 0)),
            pl.BlockSpec((1, co), lambda i: (0, 0)),
            pl.BlockSpec((9, co, co), lambda i: (0, 0, 0)),
            pl.BlockSpec((cin, co), lambda i: (0, 0)),
        ],
        out_specs=pl.BlockSpec((None, m, co), lambda i: (i, 0, 0)),
        out_shape=jax.ShapeDtypeStruct((n, m, co), jnp.float32),
        compiler_params=pltpu.CompilerParams(
            dimension_semantics=("parallel",),
            vmem_limit_bytes=_VMEM_LIMIT),
        cost_estimate=pl.CostEstimate(
            flops=2 * n * m * 9 * (cin + co) * co + 2 * n * m * cin * co,
            transcendentals=0,
            bytes_accessed=2 * n * 4 * ho * wo * cin + 4 * n * m * co),
    )(xp, s1.reshape(1, cin), b1.reshape(1, cin), wp1,
      s2.reshape(1, co), b2.reshape(1, co), wp2, wsc)

    return out.reshape(n, ho, wo, co).transpose(0, 3, 1, 2)
